# Initial kernel scaffold; baseline (speedup 1.0000x reference)
import jax
import jax.numpy as jnp
from jax.experimental import pallas as pl


def kernel(x, edge_index, t, W, b):
    raise NotImplementedError("write your pallas kernel here")


# speedup vs baseline: 7.5170x; 7.5170x over previous
import functools

import jax
import jax.numpy as jnp
from jax import lax
from jax.experimental import pallas as pl
from jax.experimental.pallas import tpu as pltpu
from jax.experimental.pallas import tpu_sc as plsc

N = 10000
E = 320000
C = 128
STEP = 10

NP = 10240
NCORE = 2
NSUB = 16
KCH = 79
B = 128
EP = NCORE * NSUB * KCH * B
ROWS_PER_TILE = NP // NSUB
ZROWS = 64

_mesh = plsc.VectorSubcoreMesh(core_axis_name="c", subcore_axis_name="s")


@functools.partial(
    pl.kernel,
    out_type=jax.ShapeDtypeStruct((NCORE, NP, C), jnp.float32),
    mesh=_mesh,
    scratch_types=[
        pltpu.VMEM((KCH, B), jnp.int32),
        pltpu.VMEM((KCH, B), jnp.int32),
        pltpu.VMEM((B, C), jnp.float32),
        pltpu.VMEM((ZROWS, C), jnp.float32),
        pltpu.VMEM_SHARED((NP, C), jnp.float32),
        pltpu.SemaphoreType.DMA,
    ],
)
def _sc_hop(g_hbm, row_hbm, col_hbm, out_hbm, row_v, col_v, buf, zbuf, acc, sem):
    cid = lax.axis_index("c")
    sid = lax.axis_index("s")

    pltpu.sync_copy(row_hbm.at[cid, sid], row_v)
    pltpu.sync_copy(col_hbm.at[cid, sid], col_v)

    def zero_row(i, _):
        for k in range(C // 16):
            zbuf[i, pl.ds(k * 16, 16)] = jnp.zeros((16,), jnp.float32)
        return 0
    lax.fori_loop(0, ZROWS, zero_row, 0)

    def zero_acc(j, _):
        pltpu.sync_copy(zbuf, acc.at[pl.ds(sid * ROWS_PER_TILE + j * ZROWS, ZROWS)])
        return 0
    lax.fori_loop(0, ROWS_PER_TILE // ZROWS, zero_acc, 0)
    plsc.subcore_barrier()

    def edge_chunk(j, _):
        pltpu.async_copy(g_hbm.at[row_v.at[j]], buf, sem).wait()
        pltpu.sync_copy(buf, acc.at[col_v.at[j]], add=True)
        return 0
    lax.fori_loop(0, KCH, edge_chunk, 0)
    plsc.subcore_barrier()

    pltpu.sync_copy(
        acc.at[pl.ds(sid * ROWS_PER_TILE, ROWS_PER_TILE)],
        out_hbm.at[cid, pl.ds(sid * ROWS_PER_TILE, ROWS_PER_TILE)],
    )


def _softmax_body(t_ref, o_ref):
    t = t_ref[...]
    m = jnp.max(t, axis=0, keepdims=True)
    e = jnp.exp(t - m)
    o_ref[...] = e / jnp.sum(e, axis=0, keepdims=True)


def _tc_softmax(t):
    return pl.pallas_call(
        _softmax_body,
        out_shape=jax.ShapeDtypeStruct((STEP, C), jnp.float32),
    )(t)


_RB = 1024


def _init_body(p_ref, x_ref, tn0_ref, dis_ref, g_ref, y_ref):
    deg = p_ref[0] + p_ref[1] + 1.0
    dis = lax.rsqrt(deg)
    x = x_ref[...]
    dis_ref[...] = dis
    g_ref[...] = dis * x
    y_ref[...] = tn0_ref[...] * x


def _tc_init(degp, xp, tn0):
    grid = NP // _RB
    return pl.pallas_call(
        _init_body,
        grid=(grid,),
        in_specs=[
            pl.BlockSpec((NCORE, _RB, C), lambda i: (0, i, 0)),
            pl.BlockSpec((_RB, C), lambda i: (i, 0)),
            pl.BlockSpec((1, C), lambda i: (0, 0)),
        ],
        out_specs=[
            pl.BlockSpec((_RB, C), lambda i: (i, 0)),
            pl.BlockSpec((_RB, C), lambda i: (i, 0)),
            pl.BlockSpec((_RB, C), lambda i: (i, 0)),
        ],
        out_shape=[
            jax.ShapeDtypeStruct((NP, C), jnp.float32),
            jax.ShapeDtypeStruct((NP, C), jnp.float32),
            jax.ShapeDtypeStruct((NP, C), jnp.float32),
        ],
    )(degp, xp, tn0)


def _hop_body(p_ref, g_ref, dis_ref, y_ref, tn_ref, go_ref, yo_ref):
    dis = dis_ref[...]
    h = dis * (p_ref[0] + p_ref[1] + g_ref[...])
    yo_ref[...] = y_ref[...] + tn_ref[...] * h
    go_ref[...] = dis * h


def _tc_hop(p, g, dis, y, tni):
    grid = NP // _RB
    return pl.pallas_call(
        _hop_body,
        grid=(grid,),
        in_specs=[
            pl.BlockSpec((NCORE, _RB, C), lambda i: (0, i, 0)),
            pl.BlockSpec((_RB, C), lambda i: (i, 0)),
            pl.BlockSpec((_RB, C), lambda i: (i, 0)),
            pl.BlockSpec((_RB, C), lambda i: (i, 0)),
            pl.BlockSpec((1, C), lambda i: (0, 0)),
        ],
        out_specs=[
            pl.BlockSpec((_RB, C), lambda i: (i, 0)),
            pl.BlockSpec((_RB, C), lambda i: (i, 0)),
        ],
        out_shape=[
            jax.ShapeDtypeStruct((NP, C), jnp.float32),
            jax.ShapeDtypeStruct((NP, C), jnp.float32),
        ],
    )(p, g, dis, y, tni)


_FB = 1000


def _final_body(p_ref, g_ref, dis_ref, y_ref, tn_ref, w_ref, b_ref, o_ref):
    h = dis_ref[...] * (p_ref[0] + p_ref[1] + g_ref[...])
    y = y_ref[...] + tn_ref[...] * h
    o_ref[...] = lax.dot_general(
        y, w_ref[...], (((1,), (1,)), ((), ())),
        preferred_element_type=jnp.float32,
    ) + b_ref[...]


def _tc_final(p, g, dis, y, tn9, W, b2):
    grid = N // _FB
    return pl.pallas_call(
        _final_body,
        grid=(grid,),
        in_specs=[
            pl.BlockSpec((NCORE, _FB, C), lambda i: (0, i, 0)),
            pl.BlockSpec((_FB, C), lambda i: (i, 0)),
            pl.BlockSpec((_FB, C), lambda i: (i, 0)),
            pl.BlockSpec((_FB, C), lambda i: (i, 0)),
            pl.BlockSpec((1, C), lambda i: (0, 0)),
            pl.BlockSpec((C, C), lambda i: (0, 0)),
            pl.BlockSpec((1, C), lambda i: (0, 0)),
        ],
        out_specs=pl.BlockSpec((_FB, C), lambda i: (i, 0)),
        out_shape=jax.ShapeDtypeStruct((N, C), jnp.float32),
    )(p, g, dis, y, tn9, W, b2)


def kernel(x, edge_index, t, W, b):
    row = edge_index[0]
    col = edge_index[1]
    pad = EP - E
    dummy = jnp.full((pad,), N, jnp.int32)
    rowp = jnp.concatenate([row, dummy]).reshape(NCORE, NSUB, KCH, B)
    colp = jnp.concatenate([col, dummy]).reshape(NCORE, NSUB, KCH, B)
    xp = jnp.pad(x, ((0, NP - N), (0, 0)))
    ones_g = jnp.pad(jnp.ones((N, C), jnp.float32), ((0, NP - N), (0, 0)))

    degp = _sc_hop(ones_g, rowp, colp)
    tn = _tc_softmax(t)
    dis, g, y = _tc_init(degp, xp, tn[0:1])
    for i in range(1, STEP - 1):
        p = _sc_hop(g, rowp, colp)
        g, y = _tc_hop(p, g, dis, y, tn[i:i + 1])
    p = _sc_hop(g, rowp, colp)
    return _tc_final(p, g, dis, y, tn[STEP - 1:STEP], W, b.reshape(1, C))
